# baseline (speedup 1.0000x reference)
import functools

import jax
import jax.numpy as jnp
from jax import lax
from jax.experimental import pallas as pl
from jax.experimental.pallas import tpu as pltpu
from jax.experimental.pallas import tpu_sc as plsc

NC = 2
NS = 16
NW = NC * NS
C = 48


def _make_sc_gather(n, d, dtype):
    b_per_w = n // NW
    n_full = b_per_w // C
    tail = b_per_w - n_full * C
    n_pairs = n_full // 2
    assert n_full - n_pairs * 2 == 1 and tail > 0
    mesh = plsc.VectorSubcoreMesh(core_axis_name="c", subcore_axis_name="s")

    @functools.partial(
        pl.kernel,
        out_type=jax.ShapeDtypeStruct((n, d), dtype),
        mesh=mesh,
        scratch_types=[
            pltpu.VMEM((b_per_w,), jnp.int32),
            pltpu.VMEM((C, d), dtype),
            pltpu.VMEM((C, d), dtype),
            pltpu.SemaphoreType.DMA,
            pltpu.SemaphoreType.DMA,
        ],
    )
    def gather_kernel(idx_hbm, table_hbm, out_hbm, idx_v, buf0, buf1, sem0, sem1):
        wid = lax.axis_index("s") * NC + lax.axis_index("c")
        base = wid * b_per_w

        def idx_at(j, size=C):
            return idx_v.at[pl.ds(j * C, size)]

        pltpu.sync_copy(
            idx_hbm.at[wid, 0, pl.ds(0, 128)], idx_v.at[pl.ds(0, 128)]
        )
        pltpu.async_copy(table_hbm.at[idx_at(0)], buf0, sem0)
        pltpu.sync_copy(
            idx_hbm.at[wid, 0, pl.ds(128, b_per_w - 128)],
            idx_v.at[pl.ds(128, b_per_w - 128)],
        )

        def body(p, _):
            j = p * 2
            pltpu.async_copy(table_hbm.at[idx_at(j + 1)], buf1, sem1)
            pltpu.make_async_copy(table_hbm.at[idx_at(0)], buf0, sem0).wait()
            pltpu.sync_copy(buf0, out_hbm.at[pl.ds(base + j * C, C)])
            pltpu.async_copy(table_hbm.at[idx_at(j + 2)], buf0, sem0)
            pltpu.make_async_copy(table_hbm.at[idx_at(0)], buf1, sem1).wait()
            pltpu.sync_copy(buf1, out_hbm.at[pl.ds(base + (j + 1) * C, C)])
            return ()

        lax.fori_loop(0, n_pairs, body, (), unroll=False)

        j = n_pairs * 2
        pltpu.async_copy(
            table_hbm.at[idx_at(n_full, tail)], buf1.at[pl.ds(0, tail)], sem1
        )
        pltpu.make_async_copy(table_hbm.at[idx_at(0)], buf0, sem0).wait()
        pltpu.sync_copy(buf0, out_hbm.at[pl.ds(base + j * C, C)])
        pltpu.make_async_copy(
            table_hbm.at[idx_at(n_full, tail)], buf1.at[pl.ds(0, tail)], sem1
        ).wait()
        pltpu.sync_copy(
            buf1.at[pl.ds(0, tail)],
            out_hbm.at[pl.ds(base + n_full * C, tail)],
        )

    return gather_kernel


def kernel(x, table):
    b, s = x.shape
    _, d = table.shape
    n = b * s
    idx = x.reshape(NW, 1, n // NW).astype(jnp.int32)
    out = _make_sc_gather(n, d, table.dtype)(idx, table)
    return out.reshape(b, s, d)


# speedup vs baseline: 1.0011x; 1.0011x over previous
import functools

import jax
import jax.numpy as jnp
from jax import lax
from jax.experimental import pallas as pl
from jax.experimental.pallas import tpu as pltpu
from jax.experimental.pallas import tpu_sc as plsc

NC = 2
NS = 16
NW = NC * NS
C = 48


def _make_sc_gather(n, d, dtype):
    b_per_w = n // NW
    n_full = b_per_w // C
    tail = b_per_w - n_full * C
    n_pairs = n_full // 2
    assert n_full - n_pairs * 2 == 1 and tail > 0
    mesh = plsc.VectorSubcoreMesh(core_axis_name="c", subcore_axis_name="s")

    @functools.partial(
        pl.kernel,
        out_type=jax.ShapeDtypeStruct((n, d), dtype),
        mesh=mesh,
        scratch_types=[
            pltpu.VMEM((b_per_w,), jnp.int32),
            pltpu.VMEM((C, d), dtype),
            pltpu.VMEM((C, d), dtype),
            pltpu.SemaphoreType.DMA,
            pltpu.SemaphoreType.DMA,
        ],
    )
    def gather_kernel(idx_hbm, table_hbm, out_hbm, idx_v, buf0, buf1, sem0, sem1):
        wid = lax.axis_index("c") * NS + lax.axis_index("s")
        base = wid * b_per_w

        def idx_at(j, size=C):
            return idx_v.at[pl.ds(j * C, size)]

        pltpu.sync_copy(
            idx_hbm.at[wid, 0, pl.ds(0, 128)], idx_v.at[pl.ds(0, 128)]
        )
        pltpu.async_copy(table_hbm.at[idx_at(0)], buf0, sem0)
        pltpu.sync_copy(
            idx_hbm.at[wid, 0, pl.ds(128, b_per_w - 128)],
            idx_v.at[pl.ds(128, b_per_w - 128)],
        )

        def body(p, _):
            j = p * 2
            pltpu.async_copy(table_hbm.at[idx_at(j + 1)], buf1, sem1)
            pltpu.make_async_copy(table_hbm.at[idx_at(0)], buf0, sem0).wait()
            pltpu.sync_copy(buf0, out_hbm.at[pl.ds(base + j * C, C)])
            pltpu.async_copy(table_hbm.at[idx_at(j + 2)], buf0, sem0)
            pltpu.make_async_copy(table_hbm.at[idx_at(0)], buf1, sem1).wait()
            pltpu.sync_copy(buf1, out_hbm.at[pl.ds(base + (j + 1) * C, C)])
            return ()

        lax.fori_loop(0, n_pairs, body, (), unroll=False)

        j = n_pairs * 2
        pltpu.async_copy(
            table_hbm.at[idx_at(n_full, tail)], buf1.at[pl.ds(0, tail)], sem1
        )
        pltpu.make_async_copy(table_hbm.at[idx_at(0)], buf0, sem0).wait()
        pltpu.sync_copy(buf0, out_hbm.at[pl.ds(base + j * C, C)])
        pltpu.make_async_copy(
            table_hbm.at[idx_at(n_full, tail)], buf1.at[pl.ds(0, tail)], sem1
        ).wait()
        pltpu.sync_copy(
            buf1.at[pl.ds(0, tail)],
            out_hbm.at[pl.ds(base + n_full * C, tail)],
        )

    return gather_kernel


def kernel(x, table):
    b, s = x.shape
    _, d = table.shape
    n = b * s
    idx = x.reshape(NW, 1, n // NW).astype(jnp.int32)
    out = _make_sc_gather(n, d, table.dtype)(idx, table)
    return out.reshape(b, s, d)
